# baseline (speedup 1.0000x reference)
import functools

import jax
import jax.numpy as jnp
from jax import lax
from jax.experimental import pallas as pl
from jax.experimental.pallas import tpu as pltpu
from jax.experimental.pallas import tpu_sc as plsc

_NC = 2
_NS = 16
_L = 16
_NW = _NC * _NS

_N, _D = 200000, 64
_BW = 128
_SBW = 512
_NSB = 390
_TAIL0 = _NSB * _SBW
_BLKE = _BW * _D
_WIN = _BLKE + 16
_NWORD = 50048
_BIG = (_NSB - 12 * _NW) // 2


def _worker_id():
    return lax.axis_index("s") * _NC + lax.axis_index("c")


def _assignment(wid):
    nsb = 12 + 2 * jnp.where(wid < _BIG, 1, 0)
    sb0 = wid * 12 + 2 * jnp.minimum(wid, _BIG)
    return nsb, sb0


def _byte(wv, k):
    return (wv >> (8 * k)) & 255 if k else wv & 255


def _count_body(mw_hbm, cnt_hbm, mwa, mwb, cbuf, sma, smb):
    wid = _worker_id()
    nsb, sb0 = _assignment(wid)

    def wsl(s):
        return mw_hbm.at[:, pl.ds(s * (_SBW // 4), _SBW // 4)]

    pltpu.async_copy(wsl(sb0), mwa, sma)
    pltpu.async_copy(wsl(sb0 + 1), mwb, smb)

    def csb(sidx, acc, buf, sem):
        pltpu.make_async_copy(wsl(sb0 + sidx), buf, sem).wait()

        def col(j, a):
            for g in range(8):
                wv = buf[j, pl.ds(g * _L, _L)]
                for k in range(4):
                    a = a + _byte(wv, k)
            return a

        acc = lax.fori_loop(0, _D, col, acc)

        @pl.when(sidx + 2 < nsb)
        def _():
            pltpu.async_copy(wsl(sb0 + sidx + 2), buf, sem)

        return acc

    def pair(t, acc):
        acc = csb(2 * t, acc, mwa, sma)
        acc = csb(2 * t + 1, acc, mwb, smb)
        return acc

    acc = lax.fori_loop(0, nsb // 2, pair, jnp.zeros((_L,), jnp.int32))
    cbuf[...] = acc
    pltpu.sync_copy(cbuf, cnt_hbm.at[pl.ds(wid * _L, _L)])


def _make_main_body(cap):
    def body(data_hbm, mw_hbm, imps_hbm, cnt_hbm, out_hbm,
             dbuf0, dbuf1, ibuf0, ibuf1, obuf0, obuf1, mwa, mwb,
             cntb, dtb, otb, sma, smb, sin0, sin1, sout0, sout1):
        wid = _worker_id()
        nsb, sb0 = _assignment(wid)

        pltpu.sync_copy(cnt_hbm, cntb)
        vec = jnp.zeros((_L,), jnp.int32)
        for w in range(_NW):
            vec = vec + jnp.where(w < wid, cntb[pl.ds(w * _L, _L)], 0)
        off0 = plsc.cumsum(vec)[15]
        al0 = pl.multiple_of(jnp.minimum((off0 // 8) * 8, cap), 8)

        def dsl(r0, width=_BW):
            return data_hbm.at[:, pl.ds(r0, width)]

        def osl(r0, width=_BW):
            return out_hbm.at[:, pl.ds(r0, width)]

        def isl(al):
            return imps_hbm.at[pl.ds(al, _WIN)]

        def wsl(s):
            return mw_hbm.at[:, pl.ds(s * (_SBW // 4), _SBW // 4)]

        zeros = jnp.zeros((_L,), jnp.int32)

        def wprepass(mwbuf, woff, nh, off, al):
            def pcol(j, cs):
                out = list(cs)
                for h in range(nh):
                    wv = mwbuf[j, pl.ds(woff + 16 * h, _L)]
                    for k in range(4):
                        out[4 * h + k] = out[4 * h + k] + _byte(wv, k)
                return tuple(out)

            cnts = lax.fori_loop(0, _D, pcol, (zeros,) * (4 * nh))
            rem = off - al
            bases = []
            gb = rem
            for g in range(4 * nh):
                iq = plsc.cumsum(cnts[g])
                bases.append(gb + (iq - cnts[g]))
                gb = gb + iq[15]
            return bases, al + gb

        def wmain(db, mwbuf, ib, ob, woff, nh, bases):
            def mcol(j, runs):
                out = list(runs)
                for h in range(nh):
                    wv = mwbuf[j, pl.ds(woff + 16 * h, _L)]
                    for k in range(4):
                        g = 4 * h + k
                        m = _byte(wv, k)
                        mbool = m > 0
                        sl = pl.ds(64 * h + 16 * k, _L)
                        d = db[j, sl]
                        v = plsc.load_gather(ib, [out[g]], mask=mbool)
                        ob[j, sl] = jnp.where(mbool, v, d)
                        out[g] = out[g] + m
                return tuple(out)

            lax.fori_loop(0, _D, mcol, tuple(bases))

        bufsets = ((dbuf0, ibuf0, obuf0, sin0, sout0),
                   (dbuf1, ibuf1, obuf1, sin1, sout1))

        pltpu.async_copy(wsl(sb0), mwa, sma)
        pltpu.async_copy(wsl(sb0 + 1), mwb, smb)
        pltpu.async_copy(dsl(sb0 * _SBW), dbuf0, sin0)
        pltpu.async_copy(isl(al0), ibuf0, sin0)

        def do_sb(sidx, mwbuf, msem, carry):
            off, al = carry
            al = pl.multiple_of(al, 8)
            s_g = sb0 + sidx
            pltpu.make_async_copy(wsl(s_g), mwbuf, msem).wait()
            for c in range(4):
                db, ib, ob, sin, sout = bufsets[c % 2]
                ndb, nib, nob, nsin, _ns = bufsets[(c + 1) % 2]
                r0 = s_g * _SBW + c * _BW
                pltpu.make_async_copy(dsl(r0), db, sin).wait()
                pltpu.make_async_copy(isl(al), ib, sin).wait()

                if c < 3:
                    pltpu.async_copy(dsl(r0 + _BW), ndb, nsin)
                else:
                    @pl.when(sidx + 1 < nsb)
                    def _():
                        pltpu.async_copy(dsl((s_g + 1) * _SBW), ndb, nsin)

                bases, off2 = wprepass(mwbuf, 32 * c, 2, off, al)
                al2 = pl.multiple_of(
                    jnp.minimum((off2 // 8) * 8, cap), 8)

                if c < 3:
                    pltpu.async_copy(isl(al2), nib, nsin)
                else:
                    @pl.when(sidx + 1 < nsb)
                    def _():
                        pltpu.async_copy(isl(al2), nib, nsin)

                if c >= 2:
                    pltpu.make_async_copy(ob, osl(r0), sout).wait()
                else:
                    @pl.when(sidx >= 1)
                    def _():
                        pltpu.make_async_copy(ob, osl(r0), sout).wait()

                wmain(db, mwbuf, ib, ob, 32 * c, 2, bases)
                pltpu.async_copy(ob, osl(r0), sout)
                off, al = off2, al2

            @pl.when(sidx + 2 < nsb)
            def _():
                pltpu.async_copy(wsl(s_g + 2), mwbuf, msem)

            return (off, al)

        def pair(t, carry):
            carry = do_sb(2 * t, mwa, sma, carry)
            carry = do_sb(2 * t + 1, mwb, smb, carry)
            return carry

        off_end, _al = lax.fori_loop(0, nsb // 2, pair, (off0, al0))

        pltpu.make_async_copy(obuf0, osl(0), sout0).wait()
        pltpu.make_async_copy(obuf1, osl(0), sout1).wait()

        @pl.when(wid == _NW - 1)
        def _():
            pltpu.sync_copy(mw_hbm.at[:, pl.ds(_TAIL0 // 4, 128)], mwa)
            off = off_end
            for c in range(2):
                r0 = _TAIL0 + c * _BW
                pltpu.sync_copy(dsl(r0), dbuf0)
                al = pl.multiple_of(jnp.minimum((off // 8) * 8, cap), 8)
                pltpu.sync_copy(isl(al), ibuf0)
                bases, off = wprepass(mwa, 32 * c, 2, off, al)
                wmain(dbuf0, mwa, ibuf0, obuf0, 32 * c, 2, bases)
                pltpu.sync_copy(obuf0, osl(r0))
            pltpu.sync_copy(dsl(_TAIL0 + 2 * _BW, 64), dtb)
            al = pl.multiple_of(jnp.minimum((off // 8) * 8, cap), 8)
            pltpu.sync_copy(isl(al), ibuf0)
            bases, off = wprepass(mwa, 64, 1, off, al)
            wmain(dtb, mwa, ibuf0, otb, 64, 1, bases)
            pltpu.sync_copy(otb, osl(_TAIL0 + 2 * _BW, 64))

    return body


def _pack_mask_words(mask):
    pw = jnp.left_shift(jnp.int32(1), 8 * jnp.arange(4, dtype=jnp.int32))
    mt = jnp.pad(mask.T, ((0, 0), (0, 4 * _NWORD - _N)))
    v = mt.reshape(_D, _NSB + 1, 4, 2, 4, _L).astype(jnp.int32)
    w = jnp.sum(v * pw[None, None, None, None, :, None], axis=4)
    return w.reshape(_D, _NWORD)


def kernel(data, mask, imps):
    data_t = data.T
    mask_w = _pack_mask_words(mask)

    nnz = imps.shape[0]
    if nnz >= _WIN:
        cap = ((nnz - _WIN + 7) // 8) * 8
        imps_eff = imps
    else:
        cap = 0
        imps_eff = jnp.pad(imps, (0, _WIN - nnz))

    mesh = plsc.VectorSubcoreMesh(
        core_axis_name="c", subcore_axis_name="s",
        num_cores=_NC, num_subcores=_NS,
    )
    params = pltpu.CompilerParams(needs_layout_passes=False)

    counts = functools.partial(
        pl.kernel,
        mesh=mesh,
        out_type=jax.ShapeDtypeStruct((_NW * _L,), jnp.int32),
        scratch_types=[
            pltpu.VMEM((_D, _BW), jnp.int32),
            pltpu.VMEM((_D, _BW), jnp.int32),
            pltpu.VMEM((_L,), jnp.int32),
            pltpu.SemaphoreType.DMA,
            pltpu.SemaphoreType.DMA,
        ],
        compiler_params=params,
    )(_count_body)(mask_w)

    main = functools.partial(
        pl.kernel,
        mesh=mesh,
        out_type=jax.ShapeDtypeStruct((_D, _N), jnp.float32),
        scratch_types=[
            pltpu.VMEM((_D, _BW), jnp.float32),
            pltpu.VMEM((_D, _BW), jnp.float32),
            pltpu.VMEM((_WIN,), jnp.float32),
            pltpu.VMEM((_WIN,), jnp.float32),
            pltpu.VMEM((_D, _BW), jnp.float32),
            pltpu.VMEM((_D, _BW), jnp.float32),
            pltpu.VMEM((_D, _BW), jnp.int32),
            pltpu.VMEM((_D, _BW), jnp.int32),
            pltpu.VMEM((_NW * _L,), jnp.int32),
            pltpu.VMEM((_D, 64), jnp.float32),
            pltpu.VMEM((_D, 64), jnp.float32),
            pltpu.SemaphoreType.DMA,
            pltpu.SemaphoreType.DMA,
            pltpu.SemaphoreType.DMA,
            pltpu.SemaphoreType.DMA,
            pltpu.SemaphoreType.DMA,
            pltpu.SemaphoreType.DMA,
        ],
        compiler_params=params,
    )(_make_main_body(cap))
    out_t = main(data_t, mask_w, imps_eff, counts)
    return out_t.T


# speedup vs baseline: 1.5337x; 1.5337x over previous
import functools

import jax
import jax.numpy as jnp
from jax import lax
from jax.experimental import pallas as pl
from jax.experimental.pallas import tpu as pltpu
from jax.experimental.pallas import tpu_sc as plsc

_NC = 2
_NS = 16
_L = 16
_NW = _NC * _NS

_N, _D = 200000, 64
_BW = 128
_NB = _N // _BW
_TAIL = _N - _NB * _BW
_TAIL0 = _NB * _BW
_BLKE = _BW * _D
_WIN = _BLKE + 16
_NG = _BW // _L
_BIG = (_NB - 48 * _NW) // 2


def _worker_id():
    return lax.axis_index("s") * _NC + lax.axis_index("c")


def _assignment(wid):
    nblk = 48 + 2 * jnp.where(wid < _BIG, 1, 0)
    sb0 = wid * 48 + 2 * jnp.minimum(wid, _BIG)
    return nblk, sb0


def _count_body(mask_hbm, cnt_hbm, mbuf0, mbuf1, cbuf, sem0, sem1):
    wid = _worker_id()
    nblk, sb0 = _assignment(wid)

    def msl(sb):
        return mask_hbm.at[:, pl.ds(sb * _BW, _BW)]

    pltpu.async_copy(msl(sb0), mbuf0, sem0)

    def count_one(sb, acc, mb, sem, nmb, nsem):
        pltpu.make_async_copy(msl(sb), mb, sem).wait()

        @pl.when(sb + 1 - sb0 < nblk)
        def _():
            pltpu.async_copy(msl(sb + 1), nmb, nsem)

        def col(j, a):
            for g in range(_NG):
                a = a + mb[j, pl.ds(g * _L, _L)]
            return a

        return lax.fori_loop(0, _D, col, acc)

    def pair(s, acc):
        sb = sb0 + 2 * s
        acc = count_one(sb, acc, mbuf0, sem0, mbuf1, sem1)
        acc = count_one(sb + 1, acc, mbuf1, sem1, mbuf0, sem0)
        return acc

    acc = lax.fori_loop(0, nblk // 2, pair, jnp.zeros((_L,), jnp.int32))
    cbuf[...] = acc
    pltpu.sync_copy(cbuf, cnt_hbm.at[pl.ds(wid * _L, _L)])


def _make_main_body(cap):
    def body(data_hbm, mask_hbm, imps_hbm, cnt_hbm, out_hbm,
             dbuf0, dbuf1, mbuf0, mbuf1, ibuf0, ibuf1, obuf0, obuf1,
             cntb, dtb, mtb, otb, sin0, sin1, sout0, sout1):
        wid = _worker_id()
        nblk, sb0 = _assignment(wid)

        pltpu.sync_copy(cnt_hbm, cntb)
        vec = jnp.zeros((_L,), jnp.int32)
        for w in range(_NW):
            vec = vec + jnp.where(w < wid, cntb[pl.ds(w * _L, _L)], 0)
        off0 = plsc.cumsum(vec)[15]
        al0 = pl.multiple_of(jnp.minimum((off0 // 8) * 8, cap), 8)

        def dsl(sb):
            return data_hbm.at[:, pl.ds(sb * _BW, _BW)]

        def msl(sb):
            return mask_hbm.at[:, pl.ds(sb * _BW, _BW)]

        def osl(sb):
            return out_hbm.at[:, pl.ds(sb * _BW, _BW)]

        def isl(al):
            return imps_hbm.at[pl.ds(al, _WIN)]

        pltpu.async_copy(dsl(sb0), dbuf0, sin0)
        pltpu.async_copy(msl(sb0), mbuf0, sin0)
        pltpu.async_copy(isl(al0), ibuf0, sin0)

        zeros = jnp.zeros((_L,), jnp.int32)

        def expand_block(db, mb, ib, ob, off, al, ng):
            def pcol(j, cs):
                return tuple(cs[g] + mb[j, pl.ds(g * _L, _L)]
                             for g in range(ng))

            cnts = lax.fori_loop(0, _D, pcol, (zeros,) * ng)
            rem = off - al
            bases = []
            gb = rem
            for g in range(ng):
                iq = plsc.cumsum(cnts[g])
                bases.append(gb + (iq - cnts[g]))
                gb = gb + iq[15]
            off2 = al + gb

            def mcol(j, runs):
                out = list(runs)
                for g in range(ng):
                    sl = pl.ds(g * _L, _L)
                    m = mb[j, sl]
                    mbool = m > 0
                    d = db[j, sl]
                    v = plsc.load_gather(ib, [out[g]], mask=mbool)
                    ob[j, sl] = jnp.where(mbool, v, d)
                    out[g] = out[g] + m
                return tuple(out)

            lax.fori_loop(0, _D, mcol, tuple(bases))
            return off2

        def do_block(b, off, al, db, mb, ib, ob, sin, sout,
                     ndb, nmb, nib, nsin):
            sb = sb0 + b
            pltpu.make_async_copy(dsl(sb), db, sin).wait()
            pltpu.make_async_copy(msl(sb), mb, sin).wait()
            pltpu.make_async_copy(isl(al), ib, sin).wait()

            @pl.when(b + 1 < nblk)
            def _():
                pltpu.async_copy(dsl(sb + 1), ndb, nsin)
                pltpu.async_copy(msl(sb + 1), nmb, nsin)

            def pcol(j, cs):
                return tuple(cs[g] + mb[j, pl.ds(g * _L, _L)]
                             for g in range(_NG))

            cnts = lax.fori_loop(0, _D, pcol, (zeros,) * _NG)
            rem = off - al
            bases = []
            gb = rem
            for g in range(_NG):
                iq = plsc.cumsum(cnts[g])
                bases.append(gb + (iq - cnts[g]))
                gb = gb + iq[15]
            off2 = al + gb
            al2 = pl.multiple_of(jnp.minimum((off2 // 8) * 8, cap), 8)

            @pl.when(b + 1 < nblk)
            def _():
                pltpu.async_copy(isl(al2), nib, nsin)

            @pl.when(b >= 2)
            def _():
                pltpu.make_async_copy(ob, osl(sb), sout).wait()

            def mcol(j, runs):
                out = list(runs)
                for g in range(_NG):
                    sl = pl.ds(g * _L, _L)
                    m = mb[j, sl]
                    mbool = m > 0
                    d = db[j, sl]
                    v = plsc.load_gather(ib, [out[g]], mask=mbool)
                    ob[j, sl] = jnp.where(mbool, v, d)
                    out[g] = out[g] + m
                return tuple(out)

            lax.fori_loop(0, _D, mcol, tuple(bases))
            pltpu.async_copy(ob, osl(sb), sout)
            return off2, al2

        def pair(s, carry):
            off, al = carry
            al = pl.multiple_of(al, 8)
            b = 2 * s
            off, al = do_block(b, off, al, dbuf0, mbuf0, ibuf0, obuf0,
                               sin0, sout0, dbuf1, mbuf1, ibuf1, sin1)
            al = pl.multiple_of(al, 8)
            off, al = do_block(b + 1, off, al, dbuf1, mbuf1, ibuf1, obuf1,
                               sin1, sout1, dbuf0, mbuf0, ibuf0, sin0)
            return (off, al)

        off_end, _al = lax.fori_loop(0, nblk // 2, pair, (off0, al0))

        pltpu.make_async_copy(obuf0, osl(sb0), sout0).wait()
        pltpu.make_async_copy(obuf1, osl(sb0), sout1).wait()

        @pl.when(wid == _NW - 1)
        def _():
            pltpu.sync_copy(data_hbm.at[:, pl.ds(_TAIL0, _TAIL)], dtb)
            pltpu.sync_copy(mask_hbm.at[:, pl.ds(_TAIL0, _TAIL)], mtb)
            al = pl.multiple_of(
                jnp.minimum((off_end // 8) * 8, cap), 8)
            pltpu.sync_copy(isl(al), ibuf0)
            expand_block(dtb, mtb, ibuf0, otb, off_end, al, _TAIL // _L)
            pltpu.sync_copy(otb, out_hbm.at[:, pl.ds(_TAIL0, _TAIL)])

    return body


def kernel(data, mask, imps):
    data_t = data.T
    mask_t = mask.T.astype(jnp.int32)

    nnz = imps.shape[0]
    if nnz >= _WIN:
        cap = ((nnz - _WIN + 7) // 8) * 8
        imps_eff = imps
    else:
        cap = 0
        imps_eff = jnp.pad(imps, (0, _WIN - nnz))

    mesh = plsc.VectorSubcoreMesh(
        core_axis_name="c", subcore_axis_name="s",
        num_cores=_NC, num_subcores=_NS,
    )
    params = pltpu.CompilerParams(needs_layout_passes=False)

    counts = functools.partial(
        pl.kernel,
        mesh=mesh,
        out_type=jax.ShapeDtypeStruct((_NW * _L,), jnp.int32),
        scratch_types=[
            pltpu.VMEM((_D, _BW), jnp.int32),
            pltpu.VMEM((_D, _BW), jnp.int32),
            pltpu.VMEM((_L,), jnp.int32),
            pltpu.SemaphoreType.DMA,
            pltpu.SemaphoreType.DMA,
        ],
        compiler_params=params,
    )(_count_body)(mask_t)

    main = functools.partial(
        pl.kernel,
        mesh=mesh,
        out_type=jax.ShapeDtypeStruct((_D, _N), jnp.float32),
        scratch_types=[
            pltpu.VMEM((_D, _BW), jnp.float32),
            pltpu.VMEM((_D, _BW), jnp.float32),
            pltpu.VMEM((_D, _BW), jnp.int32),
            pltpu.VMEM((_D, _BW), jnp.int32),
            pltpu.VMEM((_WIN,), jnp.float32),
            pltpu.VMEM((_WIN,), jnp.float32),
            pltpu.VMEM((_D, _BW), jnp.float32),
            pltpu.VMEM((_D, _BW), jnp.float32),
            pltpu.VMEM((_NW * _L,), jnp.int32),
            pltpu.VMEM((_D, _TAIL), jnp.float32),
            pltpu.VMEM((_D, _TAIL), jnp.int32),
            pltpu.VMEM((_D, _TAIL), jnp.float32),
            pltpu.SemaphoreType.DMA,
            pltpu.SemaphoreType.DMA,
            pltpu.SemaphoreType.DMA,
            pltpu.SemaphoreType.DMA,
        ],
        compiler_params=params,
    )(_make_main_body(cap))
    out_t = main(data_t, mask_t, imps_eff, counts)
    return out_t.T
